# Initial kernel scaffold; baseline (speedup 1.0000x reference)
import jax
import jax.numpy as jnp
from jax.experimental import pallas as pl


def kernel(input_time_series):
    raise NotImplementedError("write your pallas kernel here")


# speedup vs baseline: 20.0357x; 20.0357x over previous
import functools

import jax
import jax.numpy as jnp
from jax.experimental import pallas as pl
from jax.experimental.pallas import tpu as pltpu

WIDTH = 512
STEP = 256
G = 8


def _body(x_ref, o_ref, carry_ref, *, nb):
    j = pl.program_id(0)

    @pl.when(j < nb)
    def _main():
        a0 = x_ref[0, :, :STEP]
        o_ref[:, :STEP] = jnp.where(j == 0, a0, a0 + carry_ref[...])
        for k in range(1, G):
            o_ref[:, k * STEP:(k + 1) * STEP] = (
                x_ref[k, :, :STEP] + x_ref[k - 1, :, STEP:])
        carry_ref[...] = x_ref[G - 1, :, STEP:]

    @pl.when(j == nb)
    def _tail():
        o_ref[:, :STEP] = carry_ref[...]


def kernel(input_time_series):
    x = input_time_series
    n, c, w = x.shape
    total = (n - 1) * STEP + w
    nb = n // G
    out = pl.pallas_call(
        functools.partial(_body, nb=nb),
        grid=(nb + 1,),
        in_specs=[pl.BlockSpec((G, c, w), lambda j: (jnp.minimum(j, nb - 1), 0, 0))],
        out_specs=pl.BlockSpec((c, G * STEP), lambda j: (0, j)),
        out_shape=jax.ShapeDtypeStruct((c, total), x.dtype),
        scratch_shapes=[pltpu.VMEM((c, STEP), x.dtype)],
    )(x)
    return out
